# baseline (speedup 1.0000x reference)
import functools

import jax
import jax.numpy as jnp
from jax.experimental import pallas as pl
from jax.experimental.pallas import tpu as pltpu

_NUM_CLS = 64
_BUDGET = 10
_PAD_ROWS = 16


def _select_body(x_ref, w_ref, b_ref, t_ref, loss_ref, ncorrect_ref,
                 selidx_ref, rv_ref, ri_ref, rt_ref, p_ref, *, block_rows,
                 num_blocks):
    pid = pl.program_id(0)
    half = block_rows // 2

    @pl.when(pid == 0)
    def _init():
        rv_ref[...] = jnp.full((_PAD_ROWS, _NUM_CLS), -1.0, jnp.float32)
        ri_ref[...] = jnp.zeros((_PAD_ROWS, _NUM_CLS), jnp.int32)
        rt_ref[...] = jnp.full((_PAD_ROWS, _NUM_CLS), -1, jnp.int32)

    logits = jnp.dot(x_ref[...], w_ref[...],
                     preferred_element_type=jnp.float32) + b_ref[...]
    mrow = jnp.max(logits, axis=1, keepdims=True)
    e = jnp.exp(logits - mrow)
    probs = e / jnp.sum(e, axis=1, keepdims=True)

    pf = jnp.concatenate([probs[:half], probs[half:]], axis=1)
    p_ref[...] = pf

    thr = rv_ref[_BUDGET - 1:_BUDGET, :]
    over = pf > jnp.concatenate([thr, thr], axis=1)
    cnt = jnp.sum(over.astype(jnp.int32), axis=0, keepdims=True)
    mneed = jnp.max(jnp.minimum(cnt, _BUDGET))

    rowi = jax.lax.broadcasted_iota(jnp.int32, (half, 2 * _NUM_CLS), 0)
    lane = jax.lax.broadcasted_iota(jnp.int32, (half, 2 * _NUM_CLS), 1)
    rowi16 = jax.lax.broadcasted_iota(jnp.int32, (_PAD_ROWS, _NUM_CLS), 0)
    rmask10 = rowi16 < _BUDGET
    tcol = t_ref[...]
    tf = jnp.where(lane < _NUM_CLS,
                   jnp.broadcast_to(tcol[:half], (half, 2 * _NUM_CLS)),
                   jnp.broadcast_to(tcol[half:], (half, 2 * _NUM_CLS)))

    for r in range(_BUDGET):
        @pl.when(r < mneed)
        def _round(r=r):
            v = p_ref[...]
            best = jnp.max(v, axis=0, keepdims=True)
            oh = v == best
            frow = jnp.min(jnp.where(oh, rowi, half), axis=0, keepdims=True)
            sel = rowi == frow
            ct = jnp.sum(jnp.where(sel, tf, 0), axis=0, keepdims=True)
            p_ref[...] = jnp.where(sel, -1.0, v)
            gidx = pid * block_rows + frow
            for h in range(2):
                x = best[:, h * _NUM_CLS:(h + 1) * _NUM_CLS]
                xi = gidx[:, h * _NUM_CLS:(h + 1) * _NUM_CLS] + h * half
                xt = ct[:, h * _NUM_CLS:(h + 1) * _NUM_CLS]
                rv = rv_ref[...]
                ri = ri_ref[...]
                rt = rt_ref[...]
                stay = (rv > x) | ((rv == x) & (ri < xi))
                pos = jnp.sum((stay & rmask10).astype(jnp.int32), axis=0,
                              keepdims=True)
                rvd = jnp.concatenate([rv[:1], rv[:-1]], axis=0)
                rid = jnp.concatenate([ri[:1], ri[:-1]], axis=0)
                rtd = jnp.concatenate([rt[:1], rt[:-1]], axis=0)
                newv = jnp.where(rowi16 < pos, rv,
                                 jnp.where(rowi16 == pos, x, rvd))
                newi = jnp.where(rowi16 < pos, ri,
                                 jnp.where(rowi16 == pos, xi, rid))
                newt = jnp.where(rowi16 < pos, rt,
                                 jnp.where(rowi16 == pos, xt, rtd))
                rv_ref[...] = jnp.where(rmask10, newv, -1.0)
                ri_ref[...] = jnp.where(rmask10, newi, 0)
                rt_ref[...] = jnp.where(rmask10, newt, -1)

    @pl.when(pid == num_blocks - 1)
    def _emit():
        cls = jax.lax.broadcasted_iota(jnp.int32, (_PAD_ROWS, _NUM_CLS), 1)
        lv = jnp.log(jnp.where(rmask10, rv_ref[...], 1.0))
        loss_ref[...] = (-jnp.sum(lv) / (_NUM_CLS * _BUDGET)).reshape(1, 1)
        ncorrect_ref[...] = jnp.sum(
            jnp.where(rmask10 & (rt_ref[...] == cls), 1, 0).astype(jnp.int32)
        ).reshape(1, 1)
        selidx_ref[...] = ri_ref[...]


def _unique_body(row_ref, col_ref, out_ref):
    a = row_ref[...]
    b = col_ref[...]
    eq = b == a
    r = jax.lax.broadcasted_iota(jnp.int32, (640, 640), 0)
    c = jax.lax.broadcasted_iota(jnp.int32, (640, 640), 1)
    dup_counts = jnp.sum(jnp.where(eq & (r < c), 1, 0), axis=0)
    ndup = jnp.sum(jnp.where(dup_counts > 0, 1, 0).astype(jnp.int32))
    out_ref[...] = (640 - ndup).reshape(1, 1)


@jax.jit
def kernel(unlabeled_inputs, unlabeled_targets, W, b):
    n, d = unlabeled_inputs.shape
    num_blocks = 16
    block_rows = n // num_blocks

    select = pl.pallas_call(
        functools.partial(_select_body, block_rows=block_rows,
                          num_blocks=num_blocks),
        grid=(num_blocks,),
        in_specs=[
            pl.BlockSpec((block_rows, d), lambda i: (i, 0)),
            pl.BlockSpec((d, _NUM_CLS), lambda i: (0, 0)),
            pl.BlockSpec((1, _NUM_CLS), lambda i: (0, 0)),
            pl.BlockSpec((block_rows, 1), lambda i: (i, 0)),
        ],
        out_specs=[
            pl.BlockSpec((1, 1), lambda i: (0, 0)),
            pl.BlockSpec((1, 1), lambda i: (0, 0)),
            pl.BlockSpec((_PAD_ROWS, _NUM_CLS), lambda i: (0, 0)),
        ],
        out_shape=[
            jax.ShapeDtypeStruct((1, 1), jnp.float32),
            jax.ShapeDtypeStruct((1, 1), jnp.int32),
            jax.ShapeDtypeStruct((_PAD_ROWS, _NUM_CLS), jnp.int32),
        ],
        scratch_shapes=[
            pltpu.VMEM((_PAD_ROWS, _NUM_CLS), jnp.float32),
            pltpu.VMEM((_PAD_ROWS, _NUM_CLS), jnp.int32),
            pltpu.VMEM((_PAD_ROWS, _NUM_CLS), jnp.int32),
            pltpu.VMEM((block_rows // 2, 2 * _NUM_CLS), jnp.float32),
        ],
        compiler_params=pltpu.CompilerParams(
            dimension_semantics=("arbitrary",)),
    )
    loss2d, ncorrect2d, selidx_rc = select(
        unlabeled_inputs, W, b.reshape(1, _NUM_CLS),
        unlabeled_targets.reshape(n, 1))

    selected_idx = selidx_rc.T[:, :_BUDGET].reshape(-1)

    nuniq2d = pl.pallas_call(
        _unique_body,
        in_specs=[
            pl.BlockSpec((1, 640), lambda: (0, 0)),
            pl.BlockSpec((640, 1), lambda: (0, 0)),
        ],
        out_specs=pl.BlockSpec((1, 1), lambda: (0, 0)),
        out_shape=jax.ShapeDtypeStruct((1, 1), jnp.int32),
    )(selected_idx.reshape(1, 640), selected_idx.reshape(640, 1))

    return (loss2d[0, 0], selected_idx, ncorrect2d[0, 0], nuniq2d[0, 0])
